# speedup vs baseline: 5.9315x; 5.9315x over previous
import jax
import jax.numpy as jnp
from jax import lax
from jax.experimental import pallas as pl
from jax.experimental.pallas import tpu as pltpu
from jax.experimental.pallas import tpu_sc as plsc

N = 10000
D = 128
HOPS = 3
HID = 20
NCLUS = 16

NC = 2
NS = 16
NW = NC * NS
CHUNK = 128
N_PAD = 10240
DUMMY = N
RPT = N_PAD // NS
BLK = 1280
GRID = N_PAD // BLK

_mesh = plsc.VectorSubcoreMesh(core_axis_name="c", subcore_axis_name="s")


def _prelu(x):
    return jnp.where(x > 0, x, 0.25 * x)


def _deg_body(dst_hbm, zeros16_hbm, ones16_hbm, parts_hbm, acc, dst_v, ones_v):
    c = lax.axis_index("c")
    s = lax.axis_index("s")
    wid = s * NC + c
    cpt = dst_v.shape[0]
    pltpu.sync_copy(zeros16_hbm.at[pl.ds(s * RPT, RPT)], acc.at[pl.ds(s * RPT, RPT)])
    pltpu.sync_copy(dst_hbm.at[wid], dst_v)
    pltpu.sync_copy(ones16_hbm, ones_v)
    plsc.subcore_barrier()

    def step(j, carry):
        pltpu.sync_copy(ones_v, acc.at[dst_v.at[j]], add=True)
        return carry

    lax.fori_loop(0, cpt, step, 0, unroll=False)
    plsc.subcore_barrier()
    pltpu.sync_copy(acc.at[pl.ds(s * RPT, RPT)],
                    parts_hbm.at[c, pl.ds(s * RPT, RPT)])


def _hop_body(g_hbm, zeros_hbm, src_hbm, dst_hbm, parts_hbm,
              acc, src_v, dst_v, buf, sem):
    c = lax.axis_index("c")
    s = lax.axis_index("s")
    wid = s * NC + c
    cpt = src_v.shape[0]

    @pl.when(c == 0)
    def _():
        pltpu.sync_copy(g_hbm.at[pl.ds(s * RPT, RPT)], acc.at[pl.ds(s * RPT, RPT)])

    @pl.when(c != 0)
    def _():
        pltpu.sync_copy(zeros_hbm.at[pl.ds(s * RPT, RPT)],
                        acc.at[pl.ds(s * RPT, RPT)])

    pltpu.sync_copy(src_hbm.at[wid], src_v)
    pltpu.sync_copy(dst_hbm.at[wid], dst_v)
    plsc.subcore_barrier()

    def step(j, carry):
        pltpu.async_copy(g_hbm.at[src_v.at[j]], buf, sem).wait()
        pltpu.sync_copy(buf, acc.at[dst_v.at[j]], add=True)
        return carry

    lax.fori_loop(0, cpt, step, 0, unroll=False)
    plsc.subcore_barrier()
    pltpu.sync_copy(acc.at[pl.ds(s * RPT, RPT)],
                    parts_hbm.at[c, pl.ds(s * RPT, RPT)])


def _make_deg(cpt):
    return pl.kernel(
        _deg_body,
        out_type=jax.ShapeDtypeStruct((NC, N_PAD, 16), jnp.float32),
        mesh=_mesh,
        scratch_types=[
            pltpu.VMEM_SHARED((N_PAD, 16), jnp.float32),
            pltpu.VMEM((cpt, CHUNK), jnp.int32),
            pltpu.VMEM((CHUNK, 16), jnp.float32),
        ],
    )


def _make_hop(cpt):
    return pl.kernel(
        _hop_body,
        out_type=jax.ShapeDtypeStruct((NC, N_PAD, D), jnp.float32),
        mesh=_mesh,
        scratch_types=[
            pltpu.VMEM_SHARED((N_PAD, D), jnp.float32),
            pltpu.VMEM((cpt, CHUNK), jnp.int32),
            pltpu.VMEM((cpt, CHUNK), jnp.int32),
            pltpu.VMEM((CHUNK, D), jnp.float32),
            pltpu.SemaphoreType.DMA,
        ],
    )


def _norm_body(degp_ref, feat_ref, norm_ref, g0_ref):
    degp = degp_ref[...]
    deg = jnp.sum(degp, axis=(0, 2)) * (1.0 / 16.0)
    nrm = lax.rsqrt(1.0 + deg)
    nb = jnp.broadcast_to(nrm[:, None], (BLK, D))
    norm_ref[...] = nb
    g0_ref[...] = feat_ref[...] * nb


def _combine_body(parts_ref, norm_ref, h_ref, g_ref):
    t = parts_ref[0] + parts_ref[1]
    nb = norm_ref[...]
    h = t * nb
    h_ref[...] = h
    g_ref[...] = h * nb


def _mlp_body(h0_ref, h1_ref, h2_ref, h3_ref,
              w1_ref, b1_ref, w2_ref, b2_ref, w3_ref, b3_ref,
              p1_ref, pb1_ref, p2_ref, pb2_ref, p3_ref, pb3_ref, out_ref):
    hs = (h0_ref, h1_ref, h2_ref, h3_ref)
    acc = jnp.zeros((BLK, D), jnp.float32)
    for i in range(HOPS + 1):
        z = _prelu(jnp.dot(hs[i][...], w1_ref[i],
                           preferred_element_type=jnp.float32) + b1_ref[i])
        z = _prelu(jnp.dot(z, w2_ref[i],
                           preferred_element_type=jnp.float32) + b2_ref[i])
        z = jnp.dot(z, w3_ref[i], preferred_element_type=jnp.float32) + b3_ref[i]
        u = _prelu(z)
        acc = acc + jnp.dot(u, p1_ref[i], preferred_element_type=jnp.float32)
    o = _prelu(acc + pb1_ref[...])
    o = _prelu(jnp.dot(o, p2_ref[...], preferred_element_type=jnp.float32)
               + pb2_ref[...])
    o = jnp.dot(o, p3_ref[...], preferred_element_type=jnp.float32) + pb3_ref[...]
    out_ref[...] = o


def _row_spec(shape):
    nd = len(shape)
    return pl.BlockSpec((shape[0] // GRID,) + shape[1:],
                        lambda i: (i,) + (0,) * (nd - 1))


def _full_spec(shape):
    nd = len(shape)
    return pl.BlockSpec(shape, lambda i: (0,) * nd)


def kernel(features, edge_index, W1, b1, W2, b2, W3, b3, P1, pb1, P2, pb2, P3, pb3):
    E = edge_index.shape[1]
    cpt = -(-E // (NW * CHUNK))
    e_pad = NW * cpt * CHUNK

    src = edge_index[0]
    dst = edge_index[1]
    dst_m = jnp.where(src == dst, DUMMY, dst)
    pad = e_pad - E
    src_p = jnp.concatenate([src, jnp.zeros((pad,), jnp.int32)])
    dst_p = jnp.concatenate([dst_m, jnp.full((pad,), DUMMY, jnp.int32)])
    src3 = src_p.reshape(NW, cpt, CHUNK)
    dst3 = dst_p.reshape(NW, cpt, CHUNK)

    feats_pad = jnp.zeros((N_PAD, D), jnp.float32).at[:N].set(features)
    zeros2d = jnp.zeros((N_PAD, D), jnp.float32)
    zeros16 = jnp.zeros((N_PAD, 16), jnp.float32)
    ones16 = jnp.ones((CHUNK, 16), jnp.float32)

    deg_parts = _make_deg(cpt)(dst3, zeros16, ones16)

    norm_b, g = pl.pallas_call(
        _norm_body,
        grid=(GRID,),
        in_specs=[pl.BlockSpec((NC, BLK, 16), lambda i: (0, i, 0)),
                  _row_spec((N_PAD, D))],
        out_specs=[_row_spec((N_PAD, D)), _row_spec((N_PAD, D))],
        out_shape=[jax.ShapeDtypeStruct((N_PAD, D), jnp.float32),
                   jax.ShapeDtypeStruct((N_PAD, D), jnp.float32)],
    )(deg_parts, feats_pad)

    hop = _make_hop(cpt)
    combine = pl.pallas_call(
        _combine_body,
        grid=(GRID,),
        in_specs=[pl.BlockSpec((NC, BLK, D), lambda i: (0, i, 0)),
                  _row_spec((N_PAD, D))],
        out_specs=[_row_spec((N_PAD, D)), _row_spec((N_PAD, D))],
        out_shape=[jax.ShapeDtypeStruct((N_PAD, D), jnp.float32),
                   jax.ShapeDtypeStruct((N_PAD, D), jnp.float32)],
    )

    hs = [feats_pad]
    for _ in range(HOPS):
        parts = hop(g, zeros2d, src3, dst3)
        h, g = combine(parts, norm_b)
        hs.append(h)

    W1p = jnp.zeros((HOPS + 1, D, D), jnp.float32).at[:, :, :HID].set(W1)
    b1p = jnp.zeros((HOPS + 1, D), jnp.float32).at[:, :HID].set(b1)
    W2p = jnp.zeros((HOPS + 1, D, D), jnp.float32).at[:, :HID, :HID].set(W2)
    b2p = jnp.zeros((HOPS + 1, D), jnp.float32).at[:, :HID].set(b2)
    W3p = jnp.zeros((HOPS + 1, D, D), jnp.float32).at[:, :HID, :HID].set(W3)
    b3p = jnp.zeros((HOPS + 1, D), jnp.float32).at[:, :HID].set(b3)
    P1p = jnp.zeros((HOPS + 1, D, D), jnp.float32).at[:, :HID, :HID].set(
        P1.reshape(HOPS + 1, HID, HID))
    pb1p = jnp.zeros((1, D), jnp.float32).at[0, :HID].set(pb1)
    P2p = jnp.zeros((D, D), jnp.float32).at[:HID, :HID].set(P2)
    pb2p = jnp.zeros((1, D), jnp.float32).at[0, :HID].set(pb2)
    P3p = jnp.zeros((D, D), jnp.float32).at[:HID, :NCLUS].set(P3)
    pb3p = jnp.zeros((1, D), jnp.float32).at[0, :NCLUS].set(pb3)

    out_pad = pl.pallas_call(
        _mlp_body,
        grid=(GRID,),
        in_specs=[_row_spec((N_PAD, D))] * 4 + [
            _full_spec((HOPS + 1, D, D)), _full_spec((HOPS + 1, D)),
            _full_spec((HOPS + 1, D, D)), _full_spec((HOPS + 1, D)),
            _full_spec((HOPS + 1, D, D)), _full_spec((HOPS + 1, D)),
            _full_spec((HOPS + 1, D, D)), _full_spec((1, D)),
            _full_spec((D, D)), _full_spec((1, D)),
            _full_spec((D, D)), _full_spec((1, D)),
        ],
        out_specs=_row_spec((N_PAD, D)),
        out_shape=jax.ShapeDtypeStruct((N_PAD, D), jnp.float32),
    )(hs[0], hs[1], hs[2], hs[3],
      W1p, b1p, W2p, b2p, W3p, b3p, P1p, pb1p, P2p, pb2p, P3p, pb3p)

    return out_pad[:N, :NCLUS]
